# baseline (speedup 1.0000x reference)
import functools

import jax
import jax.numpy as jnp
from jax import lax
from jax.experimental import pallas as pl
from jax.experimental.pallas import tpu as pltpu
from jax.experimental.pallas import tpu_sc as plsc

_VOCAB = 1000000
_EMB = 128
_BATCH = 16384
_SEQ = 200
_SEQ_PAD = 208
_MAX_NORM = 1.0


_QC = 50000
_QGRID = _VOCAB // _QC


def _q_body(u_ref, e_ref, q_ref):
    e = e_ref[...]
    u = u_ref[...]
    dn = (((1,), (1,)), ((), ()))
    dd = lax.dot_general(u, e, dn, precision=lax.Precision.DEFAULT,
                         preferred_element_type=jnp.float32)
    ones = jnp.ones((1, _EMB), jnp.float32)
    sq = lax.dot_general(ones, e * e, dn, precision=lax.Precision.DEFAULT,
                         preferred_element_type=jnp.float32)
    norm = jnp.sqrt(sq)
    scale = jnp.where(norm > _MAX_NORM, _MAX_NORM / (norm + 1e-7), 1.0)
    q_ref[...] = (dd * scale).reshape(1, 1, _QC)


def _compute_q(emb_weight, u):
    return pl.pallas_call(
        _q_body,
        grid=(_QGRID,),
        in_specs=[
            pl.BlockSpec((1, _EMB), lambda i: (0, 0)),
            pl.BlockSpec((_QC, _EMB), lambda i: (i, 0)),
        ],
        out_specs=pl.BlockSpec((1, 1, _QC), lambda i: (i, 0, 0)),
        out_shape=jax.ShapeDtypeStruct((_QGRID, 1, _QC), jnp.float32),
    )(u, emb_weight)


_NC = 2
_NS = 16
_NW = _NC * _NS
_RPW = _BATCH // _NW
_GC = 128
_NGROUP = _RPW // _GC
_NV = _GC // 16


def _sc_body(q_hbm, xt_hbm, db_hbm, out_hbm,
             idx_v, vals_v, p0_v, p1_v, db_v, gsem):
    wid = lax.axis_index("s") * _NC + lax.axis_index("c")
    base_row = wid * _RPW
    pltpu.sync_copy(db_hbm, db_v)
    db = db_v[...]
    inv = jnp.float32(1.0 / _SEQ)

    def group(gi, carry):
        row0 = base_row + gi * _GC
        pltpu.sync_copy(xt_hbm.at[:, pl.ds(row0, _GC)], idx_v)

        def fire(j, c):
            pltpu.async_copy(q_hbm.at[idx_v.at[j]], vals_v.at[j], gsem)
            return c

        lax.fori_loop(0, _SEQ, fire, 0)

        def drain(j, c):
            pltpu.make_async_copy(q_hbm.at[idx_v.at[j]], vals_v.at[j],
                                  gsem).wait()
            return c

        lax.fori_loop(0, _SEQ, drain, 0)

        def accum(j, accs):
            return tuple(accs[v] + vals_v[j, pl.ds(16 * v, 16)]
                         for v in range(_NV))

        accs = lax.fori_loop(
            0, _SEQ, accum,
            tuple(jnp.zeros((16,), jnp.float32) for _ in range(_NV)))
        for v in range(_NV):
            d = accs[v] * inv + db
            p0_v[pl.ds(16 * v, 16)] = 1.0 / (1.0 + jnp.exp(d))
            p1_v[pl.ds(16 * v, 16)] = 1.0 / (1.0 + jnp.exp(-d))
        pltpu.sync_copy(p0_v, out_hbm.at[0, pl.ds(row0, _GC)])
        pltpu.sync_copy(p1_v, out_hbm.at[1, pl.ds(row0, _GC)])
        return carry

    lax.fori_loop(0, _NGROUP, group, 0)


@functools.cache
def _sc_kernel():
    return pl.kernel(
        _sc_body,
        out_type=jax.ShapeDtypeStruct((2, _BATCH), jnp.float32),
        mesh=plsc.VectorSubcoreMesh(core_axis_name="c", subcore_axis_name="s",
                                    num_cores=_NC, num_subcores=_NS),
        scratch_types=[
            pltpu.VMEM((_SEQ, _GC), jnp.int32),
            pltpu.VMEM((_SEQ, _GC), jnp.float32),
            pltpu.VMEM((_GC,), jnp.float32),
            pltpu.VMEM((_GC,), jnp.float32),
            pltpu.VMEM((16,), jnp.float32),
            pltpu.SemaphoreType.DMA,
        ],
    )


def kernel(x, emb_weight, lin_w, lin_b):
    u = (lin_w[1] - lin_w[0]).reshape(1, _EMB)
    q = _compute_q(emb_weight, u).reshape(_VOCAB)
    db = jnp.full((16,), lin_b[1] - lin_b[0], jnp.float32)
    out = _sc_kernel()(q, x.T, db)
    return out.T


# speedup vs baseline: 1.3488x; 1.3488x over previous
import functools

import jax
import jax.numpy as jnp
from jax import lax
from jax.experimental import pallas as pl
from jax.experimental.pallas import tpu as pltpu
from jax.experimental.pallas import tpu_sc as plsc

_VOCAB = 1000000
_EMB = 128
_BATCH = 16384
_SEQ = 200
_SEQ_PAD = 208
_MAX_NORM = 1.0


_QC = 40000
_QGRID = _VOCAB // _QC


def _q_body(u_ref, e_ref, q_ref):
    e = e_ref[...]
    u = u_ref[...]
    dn = (((1,), (1,)), ((), ()))
    dd = lax.dot_general(u, e, dn, precision=lax.Precision.DEFAULT,
                         preferred_element_type=jnp.float32)
    ones = jnp.ones((1, _EMB), jnp.float32)
    sq = lax.dot_general(ones, e * e, dn, precision=lax.Precision.DEFAULT,
                         preferred_element_type=jnp.float32)
    norm = jnp.sqrt(sq)
    scale = jnp.where(norm > _MAX_NORM, _MAX_NORM / (norm + 1e-7), 1.0)
    q_ref[...] = (dd * scale).reshape(1, 1, _QC)


def _compute_q(emb_weight, u):
    return pl.pallas_call(
        _q_body,
        grid=(_QGRID,),
        in_specs=[
            pl.BlockSpec((1, _EMB), lambda i: (0, 0)),
            pl.BlockSpec((_QC, _EMB), lambda i: (i, 0)),
        ],
        out_specs=pl.BlockSpec((1, 1, _QC), lambda i: (i, 0, 0)),
        out_shape=jax.ShapeDtypeStruct((_QGRID, 1, _QC), jnp.float32),
    )(u, emb_weight)


_NC = 2
_NS = 16
_NW = _NC * _NS
_RPW = _BATCH // _NW
_GC = 128
_NGROUP = _RPW // _GC
_NV = _GC // 16


def _sc_body(q_hbm, xt_hbm, db_hbm, out_hbm,
             q_sh, idx_v, vals_v, p0_v, p1_v, db_v, gsem):
    sid = lax.axis_index("s")
    wid = sid * _NC + lax.axis_index("c")
    base_row = wid * _RPW

    @pl.when(sid == 0)
    def _():
        pltpu.sync_copy(q_hbm, q_sh)

    plsc.subcore_barrier()
    pltpu.sync_copy(db_hbm, db_v)
    db = db_v[...]
    inv = jnp.float32(1.0 / _SEQ)

    def group(gi, carry):
        row0 = base_row + gi * _GC
        pltpu.sync_copy(xt_hbm.at[:, pl.ds(row0, _GC)], idx_v)

        def fire(j, c):
            pltpu.async_copy(q_sh.at[idx_v.at[j]], vals_v.at[j], gsem)
            return c

        lax.fori_loop(0, _SEQ, fire, 0)

        def drain(j, c):
            pltpu.make_async_copy(q_sh.at[idx_v.at[j]], vals_v.at[j],
                                  gsem).wait()
            return c

        lax.fori_loop(0, _SEQ, drain, 0)

        def accum(j, accs):
            return tuple(accs[v] + vals_v[j, pl.ds(16 * v, 16)]
                         for v in range(_NV))

        accs = lax.fori_loop(
            0, _SEQ, accum,
            tuple(jnp.zeros((16,), jnp.float32) for _ in range(_NV)))
        for v in range(_NV):
            d = accs[v] * inv + db
            p0_v[pl.ds(16 * v, 16)] = 1.0 / (1.0 + jnp.exp(d))
            p1_v[pl.ds(16 * v, 16)] = 1.0 / (1.0 + jnp.exp(-d))
        pltpu.sync_copy(p0_v, out_hbm.at[0, pl.ds(row0, _GC)])
        pltpu.sync_copy(p1_v, out_hbm.at[1, pl.ds(row0, _GC)])
        return carry

    lax.fori_loop(0, _NGROUP, group, 0)


@functools.cache
def _sc_kernel():
    return pl.kernel(
        _sc_body,
        out_type=jax.ShapeDtypeStruct((2, _BATCH), jnp.float32),
        mesh=plsc.VectorSubcoreMesh(core_axis_name="c", subcore_axis_name="s",
                                    num_cores=_NC, num_subcores=_NS),
        scratch_types=[
            pltpu.VMEM_SHARED((_VOCAB,), jnp.float32),
            pltpu.VMEM((_SEQ, _GC), jnp.int32),
            pltpu.VMEM((_SEQ, _GC), jnp.float32),
            pltpu.VMEM((_GC,), jnp.float32),
            pltpu.VMEM((_GC,), jnp.float32),
            pltpu.VMEM((16,), jnp.float32),
            pltpu.SemaphoreType.DMA,
        ],
    )


def kernel(x, emb_weight, lin_w, lin_b):
    u = (lin_w[1] - lin_w[0]).reshape(1, _EMB)
    q = _compute_q(emb_weight, u).reshape(_VOCAB)
    db = jnp.full((16,), lin_b[1] - lin_b[0], jnp.float32)
    out = _sc_kernel()(q, x.T, db)
    return out.T
